# Initial kernel scaffold; baseline (speedup 1.0000x reference)
import jax
import jax.numpy as jnp
from jax.experimental import pallas as pl


def kernel(x, edge_index, W1, asrc1, adst1, b1, W2, asrc2, adst2, b2, W3, asrc3, adst3, b3):
    raise NotImplementedError("write your pallas kernel here")


# speedup vs baseline: 1.0615x; 1.0615x over previous
import functools

import jax
import jax.numpy as jnp
from jax.experimental import pallas as pl
from jax.experimental.pallas import tpu as pltpu


def _matmul_body(x_ref, w_ref, o_ref):
    o_ref[...] = jnp.dot(x_ref[...], w_ref[...], preferred_element_type=jnp.float32)


def _pallas_matmul(x, w, block_m=1000):
    m, k = x.shape
    n = w.shape[1]
    k_pad = (-k) % 128
    m_pad = (-m) % block_m
    if k_pad:
        x = jnp.pad(x, ((0, 0), (0, k_pad)))
        w = jnp.pad(w, ((0, k_pad), (0, 0)))
    if m_pad:
        x = jnp.pad(x, ((0, m_pad), (0, 0)))
    kp = k + k_pad
    mp = m + m_pad
    out = pl.pallas_call(
        _matmul_body,
        grid=(mp // block_m,),
        in_specs=[
            pl.BlockSpec((block_m, kp), lambda i: (i, 0)),
            pl.BlockSpec((kp, n), lambda i: (0, 0)),
        ],
        out_specs=pl.BlockSpec((block_m, n), lambda i: (i, 0)),
        out_shape=jax.ShapeDtypeStruct((mp, n), jnp.float32),
    )(x, w)
    return out[:m]


def _gat_layer(x, src, dst, W, att_src, att_dst, bias, heads, out_ch, concat):
    n = x.shape[0]
    h = _pallas_matmul(x, W).reshape(n, heads, out_ch)
    a_src = (h * att_src[None, :, :]).sum(-1)
    a_dst = (h * att_dst[None, :, :]).sum(-1)
    alpha = jax.nn.leaky_relu(a_src[src] + a_dst[dst], negative_slope=0.2)
    ex = jnp.exp(alpha)
    denom = jax.ops.segment_sum(ex, dst, num_segments=n)
    out = jax.ops.segment_sum(h[src] * ex[:, :, None], dst, num_segments=n)
    out = out / (denom + 1e-16)[:, :, None]
    if concat:
        out = out.reshape(n, heads * out_ch)
    else:
        out = out.mean(axis=1)
    return out + bias


def kernel(x, edge_index, W1, asrc1, adst1, b1, W2, asrc2, adst2, b2, W3, asrc3, adst3, b3):
    n = x.shape[0]
    loop = jnp.arange(n, dtype=edge_index.dtype)
    src = jnp.concatenate([edge_index[0], loop])
    dst = jnp.concatenate([edge_index[1], loop])
    h = jax.nn.elu(_gat_layer(x, src, dst, W1, asrc1, adst1, b1, 4, 16, True))
    h = jax.nn.elu(_gat_layer(h, src, dst, W2, asrc2, adst2, b2, 4, 16, True))
    h = jax.nn.elu(_gat_layer(h, src, dst, W3, asrc3, adst3, b3, 6, 7, False))
    return jax.nn.log_softmax(h, axis=1)


# speedup vs baseline: 45.6995x; 43.0521x over previous
import functools

import jax
import jax.numpy as jnp
from jax import lax
from jax.experimental import pallas as pl
from jax.experimental.pallas import tpu as pltpu
from jax.experimental.pallas import tpu_sc as plsc

NSC = 2
NTILE = 16
LANES = 16
K_CHUNK = 512
IDX_B = 128
IDX_N = K_CHUNK // IDX_B


def _matmul_body(x_ref, w_ref, o_ref):
    o_ref[...] = jnp.dot(x_ref[...], w_ref[...], preferred_element_type=jnp.float32)


def _pallas_matmul(x, w, block_m=1000):
    m, k = x.shape
    n = w.shape[1]
    k_pad = (-k) % 128
    m_pad = (-m) % block_m
    if k_pad:
        x = jnp.pad(x, ((0, 0), (0, k_pad)))
        w = jnp.pad(w, ((0, k_pad), (0, 0)))
    if m_pad:
        x = jnp.pad(x, ((0, m_pad), (0, 0)))
    kp, mp = k + k_pad, m + m_pad
    out = pl.pallas_call(
        _matmul_body,
        grid=(mp // block_m,),
        in_specs=[
            pl.BlockSpec((block_m, kp), lambda i: (i, 0)),
            pl.BlockSpec((kp, n), lambda i: (0, 0)),
        ],
        out_specs=pl.BlockSpec((block_m, n), lambda i: (i, 0)),
        out_shape=jax.ShapeDtypeStruct((mp, n), jnp.float32),
    )(x, w)
    return out[:m]


@functools.lru_cache(maxsize=None)
def _make_edge_kernel(n, ep_pad, ep_real, heads):
    passes = heads // NSC
    ept = ep_pad // NTILE
    g_chunks = ept // K_CHUNK
    nrows_t = n // NTILE
    mesh = plsc.VectorSubcoreMesh(core_axis_name="c", subcore_axis_name="s")

    @functools.partial(
        pl.kernel,
        mesh=mesh,
        compiler_params=pltpu.CompilerParams(use_tc_tiling_on_sc=False),
        out_type=(jax.ShapeDtypeStruct((heads, n, LANES), jnp.float32),
                  jax.ShapeDtypeStruct((heads, n), jnp.float32)),
        scratch_types=[
            pltpu.VMEM_SHARED((n, LANES), jnp.float32),
            pltpu.VMEM_SHARED((n,), jnp.float32),
            pltpu.VMEM((IDX_N, IDX_B), jnp.int32),
            pltpu.VMEM((IDX_N, IDX_B), jnp.int32),
            pltpu.VMEM((K_CHUNK, LANES), jnp.float32),
            pltpu.VMEM((K_CHUNK, LANES), jnp.float32),
            pltpu.VMEM((K_CHUNK,), jnp.float32),
            pltpu.VMEM((K_CHUNK,), jnp.float32),
            pltpu.VMEM((K_CHUNK,), jnp.float32),
            pltpu.SemaphoreType.DMA,
        ],
    )
    def ek(tab, ast, adt, src2d, dst2d, zacc, zden, acc_out, den_out,
           acc_s, den_s, srcv, dstv, rows, upd, asv, adv, exv, sem):
        c = lax.axis_index("c")
        s = lax.axis_index("s")
        r0 = s * nrows_t
        for p in range(passes):
            head = c + NSC * p
            pltpu.sync_copy(zacc.at[pl.ds(r0, nrows_t)],
                            acc_s.at[pl.ds(r0, nrows_t)])
            pltpu.sync_copy(zden.at[pl.ds(r0, nrows_t)],
                            den_s.at[pl.ds(r0, nrows_t)])
            plsc.subcore_barrier()

            def chunk(g, carry):
                row_off = s * (ept // IDX_B) + g * IDX_N
                pltpu.sync_copy(src2d.at[pl.ds(row_off, IDX_N)], srcv)
                pltpu.sync_copy(dst2d.at[pl.ds(row_off, IDX_N)], dstv)
                cps = []
                for j in range(IDX_N):
                    sl = pl.ds(j * IDX_B, IDX_B)
                    cps.append(pltpu.async_copy(
                        tab.at[head].at[srcv.at[j]], rows.at[sl], sem))
                    cps.append(pltpu.async_copy(
                        ast.at[head].at[srcv.at[j]], asv.at[sl], sem))
                    cps.append(pltpu.async_copy(
                        adt.at[head].at[dstv.at[j]], adv.at[sl], sem))
                for cp in cps:
                    cp.wait()

                def inner(i, carry2):
                    base = i * LANES
                    sl = pl.ds(base, LANES)
                    alpha = asv[sl] + adv[sl]
                    ex = jnp.exp(jnp.maximum(alpha, 0.2 * alpha))
                    glob = (s * ept + g * K_CHUNK + base
                            + lax.iota(jnp.int32, LANES))
                    ex = jnp.where(glob < ep_real, ex, 0.0)
                    exv[sl] = ex
                    for j in range(LANES):
                        r = base + j
                        sv = ex[j]
                        upd[r, pl.ds(0, LANES)] = rows[r, pl.ds(0, LANES)] * sv
                    return carry2

                lax.fori_loop(0, K_CHUNK // LANES, inner, 0)
                for j in range(IDX_N):
                    sl = pl.ds(j * IDX_B, IDX_B)
                    pltpu.sync_copy(upd.at[sl], acc_s.at[dstv.at[j]], add=True)
                    pltpu.sync_copy(exv.at[sl], den_s.at[dstv.at[j]], add=True)
                return carry

            lax.fori_loop(0, g_chunks, chunk, 0)
            plsc.subcore_barrier()
            pltpu.sync_copy(acc_s.at[pl.ds(r0, nrows_t)],
                            acc_out.at[head].at[pl.ds(r0, nrows_t)])
            pltpu.sync_copy(den_s.at[pl.ds(r0, nrows_t)],
                            den_out.at[head].at[pl.ds(r0, nrows_t)])
            plsc.subcore_barrier()

    return ek


def _gat_layer(x, src2d, dst2d, ep_real, W, att_src, att_dst, bias, heads,
               out_ch, concat):
    n = x.shape[0]
    n_pad = ((n + NTILE * 8 - 1) // (NTILE * 8)) * (NTILE * 8)
    ep_pad = src2d.shape[0] * IDX_B
    h = _pallas_matmul(x, W).reshape(n, heads, out_ch)
    a_src = (h * att_src[None, :, :]).sum(-1)
    a_dst = (h * att_dst[None, :, :]).sum(-1)
    hT = h.transpose(1, 0, 2)
    tab = jnp.pad(hT, ((0, 0), (0, n_pad - n), (0, LANES - out_ch)))
    ast = jnp.pad(a_src.T, ((0, 0), (0, n_pad - n)))
    adt = jnp.pad(a_dst.T, ((0, 0), (0, n_pad - n)))
    zacc = jnp.zeros((n_pad, LANES), jnp.float32)
    zden = jnp.zeros((n_pad,), jnp.float32)
    ek = _make_edge_kernel(n_pad, ep_pad, ep_real, heads)
    acc, den = ek(tab, ast, adt, src2d, dst2d, zacc, zden)
    num = acc[:, :n, 0:out_ch]
    den = den[:, :n]
    out = num / (den + 1e-16)[:, :, None]
    if concat:
        out = out.transpose(1, 0, 2).reshape(n, heads * out_ch)
    else:
        out = out.mean(axis=0)
    return out + bias


def kernel(x, edge_index, W1, asrc1, adst1, b1, W2, asrc2, adst2, b2, W3,
           asrc3, adst3, b3):
    n = x.shape[0]
    e = edge_index.shape[1]
    ep_real = e + n
    ep_pad = ((ep_real + NTILE * K_CHUNK - 1) // (NTILE * K_CHUNK)) * (NTILE * K_CHUNK)
    loop = jnp.arange(n, dtype=edge_index.dtype)
    padv = jnp.zeros((ep_pad - ep_real,), edge_index.dtype)
    src2d = jnp.concatenate([edge_index[0], loop, padv]).reshape(-1, IDX_B)
    dst2d = jnp.concatenate([edge_index[1], loop, padv]).reshape(-1, IDX_B)
    h = jax.nn.elu(_gat_layer(x, src2d, dst2d, ep_real, W1, asrc1, adst1, b1, 4, 16, True))
    h = jax.nn.elu(_gat_layer(h, src2d, dst2d, ep_real, W2, asrc2, adst2, b2, 4, 16, True))
    h = jax.nn.elu(_gat_layer(h, src2d, dst2d, ep_real, W3, asrc3, adst3, b3, 6, 7, False))
    return jax.nn.log_softmax(h, axis=1)


# speedup vs baseline: 46.7828x; 1.0237x over previous
import functools

import jax
import jax.numpy as jnp
from jax import lax
from jax.experimental import pallas as pl
from jax.experimental.pallas import tpu as pltpu
from jax.experimental.pallas import tpu_sc as plsc

NSC = 2
NTILE = 16
LANES = 16
K_CHUNK = 512
IDX_B = 128
IDX_N = K_CHUNK // IDX_B
BLOCK_M = 512


def _mm_tab_body(heads, x_ref, w_ref, tab_ref):
    res = jnp.dot(x_ref[...], w_ref[...], preferred_element_type=jnp.float32)
    for h in range(heads):
        tab_ref[h, :, :] = res[:, h * LANES:(h + 1) * LANES]


def _mm_tab(x, w, n_pad, heads):
    m, k = x.shape
    k_pad = (-k) % 128
    if k_pad:
        x = jnp.pad(x, ((0, 0), (0, k_pad)))
        w = jnp.pad(w, ((0, k_pad), (0, 0)))
    if n_pad != m:
        x = jnp.pad(x, ((0, n_pad - m), (0, 0)))
    kp = k + k_pad
    return pl.pallas_call(
        functools.partial(_mm_tab_body, heads),
        grid=(n_pad // BLOCK_M,),
        in_specs=[
            pl.BlockSpec((BLOCK_M, kp), lambda i: (i, 0)),
            pl.BlockSpec((kp, heads * LANES), lambda i: (0, 0)),
        ],
        out_specs=pl.BlockSpec((heads, BLOCK_M, LANES), lambda i: (0, i, 0)),
        out_shape=jax.ShapeDtypeStruct((heads, n_pad, LANES), jnp.float32),
    )(x, w)


def _mm_tab_fused_body(heads_in, heads, acc_ref, den_ref, b_ref, w_ref,
                       tab_ref):
    xs = []
    for h in range(heads_in):
        xh = (acc_ref[h] / (den_ref[h][:, None] + 1e-16)
              + b_ref[h][None, :])
        xs.append(jnp.where(xh > 0, xh, jnp.exp(xh) - 1.0))
    xb = jnp.concatenate(xs, axis=1)
    res = jnp.dot(xb, w_ref[...], preferred_element_type=jnp.float32)
    for h in range(heads):
        tab_ref[h, :, :] = res[:, h * LANES:(h + 1) * LANES]


def _mm_tab_fused(acc, den, bias, w, heads):
    heads_in, n_pad, _ = acc.shape
    return pl.pallas_call(
        functools.partial(_mm_tab_fused_body, heads_in, heads),
        grid=(n_pad // BLOCK_M,),
        in_specs=[
            pl.BlockSpec((heads_in, BLOCK_M, LANES), lambda i: (0, i, 0)),
            pl.BlockSpec((heads_in, BLOCK_M), lambda i: (0, i)),
            pl.BlockSpec((heads_in, LANES), lambda i: (0, 0)),
            pl.BlockSpec((heads_in * LANES, heads * LANES), lambda i: (0, 0)),
        ],
        out_specs=pl.BlockSpec((heads, BLOCK_M, LANES), lambda i: (0, i, 0)),
        out_shape=jax.ShapeDtypeStruct((heads, n_pad, LANES), jnp.float32),
    )(acc, den, bias, w)


@functools.lru_cache(maxsize=None)
def _make_edge_kernel(n, ep_pad, ep_real, heads):
    passes = heads // NSC
    ept = ep_pad // NTILE
    g_chunks = ept // K_CHUNK
    nrows_t = n // NTILE
    mesh = plsc.VectorSubcoreMesh(core_axis_name="c", subcore_axis_name="s")

    @functools.partial(
        pl.kernel,
        mesh=mesh,
        compiler_params=pltpu.CompilerParams(use_tc_tiling_on_sc=False),
        out_type=(jax.ShapeDtypeStruct((heads, n, LANES), jnp.float32),
                  jax.ShapeDtypeStruct((heads, n), jnp.float32)),
        scratch_types=[
            pltpu.VMEM_SHARED((n, LANES), jnp.float32),
            pltpu.VMEM_SHARED((n,), jnp.float32),
            pltpu.VMEM((IDX_N, IDX_B), jnp.int32),
            pltpu.VMEM((IDX_N, IDX_B), jnp.int32),
            pltpu.VMEM((K_CHUNK, LANES), jnp.float32),
            pltpu.VMEM((K_CHUNK, LANES), jnp.float32),
            pltpu.VMEM((K_CHUNK,), jnp.float32),
            pltpu.VMEM((K_CHUNK,), jnp.float32),
            pltpu.VMEM((K_CHUNK,), jnp.float32),
            pltpu.SemaphoreType.DMA,
        ],
    )
    def ek(tab, ast, adt, src2d, dst2d, zacc, zden, acc_out, den_out,
           acc_s, den_s, srcv, dstv, rows, upd, asv, adv, exv, sem):
        c = lax.axis_index("c")
        s = lax.axis_index("s")
        r0 = s * nrows_t
        rsl = pl.ds(r0, nrows_t)
        for p in range(passes):
            head = c + NSC * p
            pltpu.sync_copy(zacc.at[rsl], acc_s.at[rsl])
            pltpu.sync_copy(zden.at[rsl], den_s.at[rsl])
            plsc.subcore_barrier()

            def chunk(g, carry):
                row_off = s * (ept // IDX_B) + g * IDX_N
                pltpu.sync_copy(src2d.at[pl.ds(row_off, IDX_N)], srcv)
                pltpu.sync_copy(dst2d.at[pl.ds(row_off, IDX_N)], dstv)
                cps = []
                for j in range(IDX_N):
                    sl = pl.ds(j * IDX_B, IDX_B)
                    cps.append(pltpu.async_copy(
                        tab.at[head].at[srcv.at[j]], rows.at[sl], sem))
                    cps.append(pltpu.async_copy(
                        ast.at[head].at[srcv.at[j]], asv.at[sl], sem))
                    cps.append(pltpu.async_copy(
                        adt.at[head].at[dstv.at[j]], adv.at[sl], sem))
                for cp in cps:
                    cp.wait()

                def inner(i, carry2):
                    base = i * LANES
                    sl = pl.ds(base, LANES)
                    alpha = asv[sl] + adv[sl]
                    ex = jnp.exp(jnp.maximum(alpha, 0.2 * alpha))
                    glob = (s * ept + g * K_CHUNK + base
                            + lax.iota(jnp.int32, LANES))
                    ex = jnp.where(glob < ep_real, ex, 0.0)
                    exv[sl] = ex
                    for jj in range(LANES):
                        r = base + jj
                        upd[r, pl.ds(0, LANES)] = (
                            rows[r, pl.ds(0, LANES)] * ex[jj])
                    return carry2

                lax.fori_loop(0, K_CHUNK // LANES, inner, 0)
                for j in range(IDX_N):
                    sl = pl.ds(j * IDX_B, IDX_B)
                    pltpu.sync_copy(upd.at[sl], acc_s.at[dstv.at[j]], add=True)
                    pltpu.sync_copy(exv.at[sl], den_s.at[dstv.at[j]], add=True)
                return carry

            lax.fori_loop(0, g_chunks, chunk, 0)
            plsc.subcore_barrier()
            pltpu.sync_copy(acc_s.at[rsl], acc_out.at[head].at[rsl])
            pltpu.sync_copy(den_s.at[rsl], den_out.at[head].at[rsl])
            plsc.subcore_barrier()

    return ek


def _pad_weights(W, att_src, att_dst, bias, heads, out_ch):
    cpad = LANES - out_ch
    Wp = jnp.pad(W.reshape(W.shape[0], heads, out_ch), ((0, 0), (0, 0), (0, cpad)))
    ap_s = jnp.pad(att_src, ((0, 0), (0, cpad)))
    ap_d = jnp.pad(att_dst, ((0, 0), (0, cpad)))
    if bias.shape[0] == heads * out_ch:
        bp = jnp.pad(bias.reshape(heads, out_ch), ((0, 0), (0, cpad)))
    else:
        bp = None
    return Wp.reshape(W.shape[0], heads * LANES), ap_s, ap_d, bp


def _edge_phase(tab, ap_s, ap_d, src2d, dst2d, ep_real, heads):
    n_pad = tab.shape[1]
    ep_pad = src2d.shape[0] * IDX_B
    ast = (tab * ap_s[:, None, :]).sum(-1)
    adt = (tab * ap_d[:, None, :]).sum(-1)
    zacc = jnp.zeros((n_pad, LANES), jnp.float32)
    zden = jnp.zeros((n_pad,), jnp.float32)
    ek = _make_edge_kernel(n_pad, ep_pad, ep_real, heads)
    return ek(tab, ast, adt, src2d, dst2d, zacc, zden)


def kernel(x, edge_index, W1, asrc1, adst1, b1, W2, asrc2, adst2, b2, W3,
           asrc3, adst3, b3):
    n = x.shape[0]
    n_pad = ((n + BLOCK_M - 1) // BLOCK_M) * BLOCK_M
    e = edge_index.shape[1]
    ep_real = e + n
    epg = NTILE * K_CHUNK
    ep_pad = ((ep_real + epg - 1) // epg) * epg
    loop = jnp.arange(n, dtype=edge_index.dtype)
    padv = jnp.zeros((ep_pad - ep_real,), edge_index.dtype)
    src2d = jnp.concatenate([edge_index[0], loop, padv]).reshape(-1, IDX_B)
    dst2d = jnp.concatenate([edge_index[1], loop, padv]).reshape(-1, IDX_B)

    Wp1, as1, ad1, bp1 = _pad_weights(W1, asrc1, adst1, b1, 4, 16)
    Wp2, as2, ad2, bp2 = _pad_weights(W2, asrc2, adst2, b2, 4, 16)
    Wp3, as3, ad3, _ = _pad_weights(W3, asrc3, adst3, b3, 6, 7)

    tab1 = _mm_tab(x, Wp1, n_pad, 4)
    acc1, den1 = _edge_phase(tab1, as1, ad1, src2d, dst2d, ep_real, 4)
    tab2 = _mm_tab_fused(acc1, den1, bp1, Wp2, 4)
    acc2, den2 = _edge_phase(tab2, as2, ad2, src2d, dst2d, ep_real, 4)
    tab3 = _mm_tab_fused(acc2, den2, bp2, Wp3, 6)
    acc3, den3 = _edge_phase(tab3, as3, ad3, src2d, dst2d, ep_real, 6)

    out = acc3[:, :n, 0:7] / (den3[:, :n] + 1e-16)[:, :, None]
    out = out.mean(axis=0) + b3
    out = jnp.where(out > 0, out, jnp.expm1(out))
    return jax.nn.log_softmax(out, axis=1)


# speedup vs baseline: 69.6581x; 1.4890x over previous
import functools

import jax
import jax.numpy as jnp
from jax import lax
from jax.experimental import pallas as pl
from jax.experimental.pallas import tpu as pltpu
from jax.experimental.pallas import tpu_sc as plsc

NSC = 2
NTILE = 16
LANES = 16
K_CHUNK = 384
IDX_B = 128
IDX_N = K_CHUNK // IDX_B
BLOCK_M = 512


def _mm_tab_body(heads, x_ref, w_ref, tab_ref):
    res = jnp.dot(x_ref[...], w_ref[...], preferred_element_type=jnp.float32)
    for h in range(heads):
        tab_ref[h, :, :] = res[:, h * LANES:(h + 1) * LANES]


def _mm_tab(x, w, n_pad, heads):
    m, k = x.shape
    return pl.pallas_call(
        functools.partial(_mm_tab_body, heads),
        grid=(n_pad // BLOCK_M,),
        in_specs=[
            pl.BlockSpec((BLOCK_M, k), lambda i: (i, 0)),
            pl.BlockSpec((k, heads * LANES), lambda i: (0, 0)),
        ],
        out_specs=pl.BlockSpec((heads, BLOCK_M, LANES), lambda i: (0, i, 0)),
        out_shape=jax.ShapeDtypeStruct((heads, n_pad, LANES), jnp.float32),
    )(x, w)


def _mm_tab_fused_body(heads_in, heads, acc_ref, den_ref, b_ref, w_ref,
                       tab_ref):
    xs = []
    for h in range(heads_in):
        xh = (acc_ref[h] / (den_ref[h][:, None] + 1e-16)
              + b_ref[h][None, :])
        xs.append(jnp.where(xh > 0, xh, jnp.exp(xh) - 1.0))
    xb = jnp.concatenate(xs, axis=1)
    res = jnp.dot(xb, w_ref[...], preferred_element_type=jnp.float32)
    for h in range(heads):
        tab_ref[h, :, :] = res[:, h * LANES:(h + 1) * LANES]


def _mm_tab_fused(acc, den, bias, w, heads):
    heads_in, n_pad, _ = acc.shape
    return pl.pallas_call(
        functools.partial(_mm_tab_fused_body, heads_in, heads),
        grid=(n_pad // BLOCK_M,),
        in_specs=[
            pl.BlockSpec((heads_in, BLOCK_M, LANES), lambda i: (0, i, 0)),
            pl.BlockSpec((heads_in, BLOCK_M), lambda i: (0, i)),
            pl.BlockSpec((heads_in, LANES), lambda i: (0, 0)),
            pl.BlockSpec((heads_in * LANES, heads * LANES), lambda i: (0, 0)),
        ],
        out_specs=pl.BlockSpec((heads, BLOCK_M, LANES), lambda i: (0, i, 0)),
        out_shape=jax.ShapeDtypeStruct((heads, n_pad, LANES), jnp.float32),
    )(acc, den, bias, w)


@functools.lru_cache(maxsize=None)
def _make_edge_kernel(n, ep_pad, ep_real, heads):
    passes = heads // NSC
    ept = ep_pad // NTILE
    g_chunks = ept // K_CHUNK
    g2_chunks = g_chunks // 2
    nrows_t = n // NTILE
    mesh = plsc.VectorSubcoreMesh(core_axis_name="c", subcore_axis_name="s")

    idx_t = pltpu.VMEM((IDX_N, IDX_B), jnp.int32)
    row_t = pltpu.VMEM((K_CHUNK, LANES), jnp.float32)
    sca_t = pltpu.VMEM((K_CHUNK,), jnp.float32)

    @functools.partial(
        pl.kernel,
        mesh=mesh,
        compiler_params=pltpu.CompilerParams(use_tc_tiling_on_sc=False),
        out_type=(jax.ShapeDtypeStruct((heads, n, LANES), jnp.float32),
                  jax.ShapeDtypeStruct((heads, n), jnp.float32)),
        scratch_types=[
            pltpu.VMEM_SHARED((n, LANES), jnp.float32),
            pltpu.VMEM_SHARED((n,), jnp.float32),
            idx_t, idx_t, idx_t, idx_t,
            row_t, row_t,
            sca_t, sca_t, sca_t, sca_t,
            row_t,
            sca_t,
            pltpu.SemaphoreType.DMA,
            pltpu.SemaphoreType.DMA,
        ],
    )
    def ek(tab, ast, adt, src2d, dst2d, zacc, zden, acc_out, den_out,
           acc_s, den_s, srcv0, dstv0, srcv1, dstv1, rows0, rows1,
           asv0, adv0, asv1, adv1, upd, exv, gsem0, gsem1):
        c = lax.axis_index("c")
        s = lax.axis_index("s")
        r0 = s * nrows_t
        rsl = pl.ds(r0, nrows_t)
        bufs = ((srcv0, dstv0, rows0, asv0, adv0, gsem0),
                (srcv1, dstv1, rows1, asv1, adv1, gsem1))
        for p in range(passes):
            head = c + NSC * p

            def gather_copies(g, b, make_only):
                srcv, dstv, rows, asv, adv, gsem = bufs[b]
                mk = pltpu.make_async_copy if make_only else pltpu.async_copy
                cps = []
                for j in range(IDX_N):
                    sl = pl.ds(j * IDX_B, IDX_B)
                    cps.append(mk(tab.at[head].at[srcv.at[j]],
                                  rows.at[sl], gsem))
                    cps.append(mk(ast.at[head].at[srcv.at[j]],
                                  asv.at[sl], gsem))
                    cps.append(mk(adt.at[head].at[dstv.at[j]],
                                  adv.at[sl], gsem))
                return cps

            def fire(g, b):
                srcv, dstv = bufs[b][0], bufs[b][1]
                row_off = s * (ept // IDX_B) + g * IDX_N
                pltpu.sync_copy(src2d.at[pl.ds(row_off, IDX_N)], srcv)
                pltpu.sync_copy(dst2d.at[pl.ds(row_off, IDX_N)], dstv)
                gather_copies(g, b, False)

            def process(g, b):
                srcv, dstv, rows, asv, adv, gsem = bufs[b]
                for cp in gather_copies(g, b, True):
                    cp.wait()

                def inner(i, carry2):
                    base = i * LANES
                    sl = pl.ds(base, LANES)
                    alpha = asv[base // IDX_B, pl.ds(base % IDX_B, LANES)] \
                        if False else asv[sl]
                    alpha = asv[sl] + adv[sl]
                    ex = jnp.exp(jnp.maximum(alpha, 0.2 * alpha))
                    glob = (s * ept + g * K_CHUNK + base
                            + lax.iota(jnp.int32, LANES))
                    ex = jnp.where(glob < ep_real, ex, 0.0)
                    exv[sl] = ex
                    for jj in range(LANES):
                        r = base + jj
                        upd[r, pl.ds(0, LANES)] = (
                            rows[r, pl.ds(0, LANES)] * ex[jj])
                    return carry2

                lax.fori_loop(0, K_CHUNK // LANES, inner, 0)
                for j in range(IDX_N):
                    sl = pl.ds(j * IDX_B, IDX_B)
                    pltpu.sync_copy(upd.at[sl], acc_s.at[dstv.at[j]], add=True)
                    pltpu.sync_copy(exv.at[sl], den_s.at[dstv.at[j]], add=True)

            pltpu.sync_copy(zacc.at[rsl], acc_s.at[rsl])
            pltpu.sync_copy(zden.at[rsl], den_s.at[rsl])
            plsc.subcore_barrier()

            fire(0, 0)
            fire(1, 1)

            def chunk2(g2, carry):
                ga = 2 * g2
                process(ga, 0)

                @pl.when(g2 < g2_chunks - 1)
                def _():
                    fire(ga + 2, 0)

                process(ga + 1, 1)

                @pl.when(g2 < g2_chunks - 1)
                def _():
                    fire(ga + 3, 1)

                return carry

            lax.fori_loop(0, g2_chunks, chunk2, 0)
            plsc.subcore_barrier()
            pltpu.sync_copy(acc_s.at[rsl], acc_out.at[head].at[rsl])
            pltpu.sync_copy(den_s.at[rsl], den_out.at[head].at[rsl])
            plsc.subcore_barrier()

    return ek


def _pad_weights(W, att_src, att_dst, bias, heads, out_ch):
    cpad = LANES - out_ch
    Wp = jnp.pad(W.reshape(W.shape[0], heads, out_ch), ((0, 0), (0, 0), (0, cpad)))
    ap_s = jnp.pad(att_src, ((0, 0), (0, cpad)))
    ap_d = jnp.pad(att_dst, ((0, 0), (0, cpad)))
    if bias.shape[0] == heads * out_ch:
        bp = jnp.pad(bias.reshape(heads, out_ch), ((0, 0), (0, cpad)))
    else:
        bp = None
    return Wp.reshape(W.shape[0], heads * LANES), ap_s, ap_d, bp


def _edge_phase(tab, ap_s, ap_d, src2d, dst2d, ep_real, heads):
    n_pad = tab.shape[1]
    ep_pad = src2d.shape[0] * IDX_B
    ast = (tab * ap_s[:, None, :]).sum(-1)
    adt = (tab * ap_d[:, None, :]).sum(-1)
    zacc = jnp.zeros((n_pad, LANES), jnp.float32)
    zden = jnp.zeros((n_pad,), jnp.float32)
    ek = _make_edge_kernel(n_pad, ep_pad, ep_real, heads)
    return ek(tab, ast, adt, src2d, dst2d, zacc, zden)


def kernel(x, edge_index, W1, asrc1, adst1, b1, W2, asrc2, adst2, b2, W3,
           asrc3, adst3, b3):
    n = x.shape[0]
    n_pad = ((n + BLOCK_M - 1) // BLOCK_M) * BLOCK_M
    e = edge_index.shape[1]
    ep_real = e + n
    epg = NTILE * K_CHUNK * 2
    ep_pad = ((ep_real + epg - 1) // epg) * epg
    loop = jnp.arange(n, dtype=edge_index.dtype)
    padv = jnp.zeros((ep_pad - ep_real,), edge_index.dtype)
    src2d = jnp.concatenate([edge_index[0], loop, padv]).reshape(-1, IDX_B)
    dst2d = jnp.concatenate([edge_index[1], loop, padv]).reshape(-1, IDX_B)

    Wp1, as1, ad1, bp1 = _pad_weights(W1, asrc1, adst1, b1, 4, 16)
    Wp2, as2, ad2, bp2 = _pad_weights(W2, asrc2, adst2, b2, 4, 16)
    Wp3, as3, ad3, _ = _pad_weights(W3, asrc3, adst3, b3, 6, 7)

    tab1 = _mm_tab(x, Wp1, n_pad, 4)
    acc1, den1 = _edge_phase(tab1, as1, ad1, src2d, dst2d, ep_real, 4)
    tab2 = _mm_tab_fused(acc1, den1, bp1, Wp2, 4)
    acc2, den2 = _edge_phase(tab2, as2, ad2, src2d, dst2d, ep_real, 4)
    tab3 = _mm_tab_fused(acc2, den2, bp2, Wp3, 6)
    acc3, den3 = _edge_phase(tab3, as3, ad3, src2d, dst2d, ep_real, 6)

    out = acc3[:, :n, 0:7] / (den3[:, :n] + 1e-16)[:, :, None]
    out = out.mean(axis=0) + b3
    out = jnp.where(out > 0, out, jnp.expm1(out))
    return jax.nn.log_softmax(out, axis=1)


# speedup vs baseline: 73.4258x; 1.0541x over previous
import functools

import jax
import jax.numpy as jnp
from jax import lax
from jax.experimental import pallas as pl
from jax.experimental.pallas import tpu as pltpu
from jax.experimental.pallas import tpu_sc as plsc

NSC = 2
NTILE = 16
LANES = 16
K_CHUNK = 384
IDX_B = 128
IDX_N = K_CHUNK // IDX_B
BLOCK_M = 512


def _mm_tab_body(heads, x_ref, w_ref, tab_ref):
    res = jnp.dot(x_ref[...], w_ref[...], preferred_element_type=jnp.float32)
    for h in range(heads):
        tab_ref[h, :, :] = res[:, h * LANES:(h + 1) * LANES]


def _mm_tab(x, w, n_pad, heads):
    m, k = x.shape
    return pl.pallas_call(
        functools.partial(_mm_tab_body, heads),
        grid=(n_pad // BLOCK_M,),
        in_specs=[
            pl.BlockSpec((BLOCK_M, k), lambda i: (i, 0)),
            pl.BlockSpec((k, heads * LANES), lambda i: (0, 0)),
        ],
        out_specs=pl.BlockSpec((heads, BLOCK_M, LANES), lambda i: (0, i, 0)),
        out_shape=jax.ShapeDtypeStruct((heads, n_pad, LANES), jnp.float32),
    )(x, w)


def _mm_tab_fused_body(heads_in, heads, acc_ref, den_ref, b_ref, w_ref,
                       tab_ref):
    xs = []
    for h in range(heads_in):
        xh = (acc_ref[h] / (den_ref[h][:, None] + 1e-16)
              + b_ref[h][None, :])
        xs.append(jnp.where(xh > 0, xh, jnp.exp(xh) - 1.0))
    xb = jnp.concatenate(xs, axis=1)
    res = jnp.dot(xb, w_ref[...], preferred_element_type=jnp.float32)
    for h in range(heads):
        tab_ref[h, :, :] = res[:, h * LANES:(h + 1) * LANES]


def _mm_tab_fused(acc, den, bias, w, heads):
    heads_in, n_pad, _ = acc.shape
    return pl.pallas_call(
        functools.partial(_mm_tab_fused_body, heads_in, heads),
        grid=(n_pad // BLOCK_M,),
        in_specs=[
            pl.BlockSpec((heads_in, BLOCK_M, LANES), lambda i: (0, i, 0)),
            pl.BlockSpec((heads_in, BLOCK_M), lambda i: (0, i)),
            pl.BlockSpec((heads_in, LANES), lambda i: (0, 0)),
            pl.BlockSpec((heads_in * LANES, heads * LANES), lambda i: (0, 0)),
        ],
        out_specs=pl.BlockSpec((heads, BLOCK_M, LANES), lambda i: (0, i, 0)),
        out_shape=jax.ShapeDtypeStruct((heads, n_pad, LANES), jnp.float32),
    )(acc, den, bias, w)


@functools.lru_cache(maxsize=None)
def _make_edge_kernel(n, ep_pad, ep_real, heads):
    passes = heads // NSC
    ept = ep_pad // NTILE
    g_chunks = ept // K_CHUNK
    g2_chunks = g_chunks // 2
    nrows_t = n // NTILE
    mesh = plsc.VectorSubcoreMesh(core_axis_name="c", subcore_axis_name="s")

    idx_t = pltpu.VMEM((IDX_N, IDX_B), jnp.int32)
    row_t = pltpu.VMEM((K_CHUNK, LANES), jnp.float32)
    sca_t = pltpu.VMEM((K_CHUNK,), jnp.float32)

    @functools.partial(
        pl.kernel,
        mesh=mesh,
        compiler_params=pltpu.CompilerParams(use_tc_tiling_on_sc=False),
        out_type=(jax.ShapeDtypeStruct((heads, n, LANES), jnp.float32),
                  jax.ShapeDtypeStruct((heads, n), jnp.float32)),
        scratch_types=[
            pltpu.VMEM_SHARED((n, LANES), jnp.float32),
            pltpu.VMEM_SHARED((n,), jnp.float32),
            idx_t, idx_t, idx_t, idx_t,
            row_t, row_t,
            sca_t, sca_t, sca_t, sca_t,
            row_t,
            sca_t,
            pltpu.SemaphoreType.DMA,
            pltpu.SemaphoreType.DMA,
        ],
    )
    def ek(tab, ast, adt, src2d, dst2d, zacc, zden, acc_out, den_out,
           acc_s, den_s, srcv0, dstv0, srcv1, dstv1, rows0, rows1,
           asv0, adv0, asv1, adv1, upd, exv, gsem0, gsem1):
        c = lax.axis_index("c")
        s = lax.axis_index("s")
        r0 = s * nrows_t
        rsl = pl.ds(r0, nrows_t)
        bufs = ((srcv0, dstv0, rows0, asv0, adv0, gsem0),
                (srcv1, dstv1, rows1, asv1, adv1, gsem1))
        for p in range(passes):
            head = c + NSC * p

            def gather_copies(g, b, make_only):
                srcv, dstv, rows, asv, adv, gsem = bufs[b]
                mk = pltpu.make_async_copy if make_only else pltpu.async_copy
                cps = []
                for j in range(IDX_N):
                    sl = pl.ds(j * IDX_B, IDX_B)
                    cps.append(mk(tab.at[head].at[srcv.at[j]],
                                  rows.at[sl], gsem))
                    cps.append(mk(ast.at[head].at[srcv.at[j]],
                                  asv.at[sl], gsem))
                    cps.append(mk(adt.at[head].at[dstv.at[j]],
                                  adv.at[sl], gsem))
                return cps

            def fire(g, b):
                srcv, dstv = bufs[b][0], bufs[b][1]
                row_off = s * (ept // IDX_B) + g * IDX_N
                pltpu.sync_copy(src2d.at[pl.ds(row_off, IDX_N)], srcv)
                pltpu.sync_copy(dst2d.at[pl.ds(row_off, IDX_N)], dstv)
                gather_copies(g, b, False)

            def process(g, b):
                srcv, dstv, rows, asv, adv, gsem = bufs[b]
                for cp in gather_copies(g, b, True):
                    cp.wait()

                def inner(i, carry2):
                    base = i * LANES
                    sl = pl.ds(base, LANES)
                    alpha = asv[sl] + adv[sl]
                    ex = jnp.exp(jnp.maximum(alpha, 0.2 * alpha))
                    exv[sl] = ex
                    for jj in range(LANES):
                        r = base + jj
                        upd[r, pl.ds(0, LANES)] = (
                            rows[r, pl.ds(0, LANES)] * ex[jj])
                    return carry2

                lax.fori_loop(0, K_CHUNK // LANES, inner, 0)
                cps2 = []
                for j in range(IDX_N):
                    sl = pl.ds(j * IDX_B, IDX_B)
                    cps2.append(pltpu.async_copy(
                        upd.at[sl], acc_s.at[dstv.at[j]], add=True, sem=gsem))
                    pltpu.sync_copy(exv.at[sl], den_s.at[dstv.at[j]], add=True)
                for cp in cps2:
                    cp.wait()

            pltpu.sync_copy(zacc.at[rsl], acc_s.at[rsl])
            pltpu.sync_copy(zden.at[rsl], den_s.at[rsl])
            plsc.subcore_barrier()

            fire(0, 0)
            fire(1, 1)

            def chunk2(g2, carry):
                ga = 2 * g2
                process(ga, 0)

                @pl.when(g2 < g2_chunks - 1)
                def _():
                    fire(ga + 2, 0)

                process(ga + 1, 1)

                @pl.when(g2 < g2_chunks - 1)
                def _():
                    fire(ga + 3, 1)

                return carry

            lax.fori_loop(0, g2_chunks, chunk2, 0)
            plsc.subcore_barrier()
            pltpu.sync_copy(acc_s.at[rsl], acc_out.at[head].at[rsl])
            pltpu.sync_copy(den_s.at[rsl], den_out.at[head].at[rsl])
            plsc.subcore_barrier()

    return ek


def _pad_weights(W, att_src, att_dst, bias, heads, out_ch):
    cpad = LANES - out_ch
    Wp = jnp.pad(W.reshape(W.shape[0], heads, out_ch), ((0, 0), (0, 0), (0, cpad)))
    ap_s = jnp.pad(att_src, ((0, 0), (0, cpad)))
    ap_d = jnp.pad(att_dst, ((0, 0), (0, cpad)))
    if bias.shape[0] == heads * out_ch:
        bp = jnp.pad(bias.reshape(heads, out_ch), ((0, 0), (0, cpad)))
    else:
        bp = None
    return Wp.reshape(W.shape[0], heads * LANES), ap_s, ap_d, bp


def _edge_phase(tab, ap_s, ap_d, src2d, dst2d, ep_real, heads):
    n_pad = tab.shape[1]
    ep_pad = src2d.shape[0] * IDX_B
    ast = (tab * ap_s[:, None, :]).sum(-1)
    adt = (tab * ap_d[:, None, :]).sum(-1)
    zacc = jnp.zeros((n_pad, LANES), jnp.float32)
    zden = jnp.zeros((n_pad,), jnp.float32)
    ek = _make_edge_kernel(n_pad, ep_pad, ep_real, heads)
    return ek(tab, ast, adt, src2d, dst2d, zacc, zden)


def kernel(x, edge_index, W1, asrc1, adst1, b1, W2, asrc2, adst2, b2, W3,
           asrc3, adst3, b3):
    n = x.shape[0]
    n_pad = ((n + BLOCK_M - 1) // BLOCK_M) * BLOCK_M
    e = edge_index.shape[1]
    ep_real = e + n
    epg = NTILE * K_CHUNK * 2
    ep_pad = ((ep_real + epg - 1) // epg) * epg
    loop = jnp.arange(n, dtype=edge_index.dtype)
    padv = n + (jnp.arange(ep_pad - ep_real, dtype=edge_index.dtype)
                % (n_pad - n))
    src2d = jnp.concatenate([edge_index[0], loop, padv]).reshape(-1, IDX_B)
    dst2d = jnp.concatenate([edge_index[1], loop, padv]).reshape(-1, IDX_B)

    Wp1, as1, ad1, bp1 = _pad_weights(W1, asrc1, adst1, b1, 4, 16)
    Wp2, as2, ad2, bp2 = _pad_weights(W2, asrc2, adst2, b2, 4, 16)
    Wp3, as3, ad3, _ = _pad_weights(W3, asrc3, adst3, b3, 6, 7)

    tab1 = _mm_tab(x, Wp1, n_pad, 4)
    acc1, den1 = _edge_phase(tab1, as1, ad1, src2d, dst2d, ep_real, 4)
    tab2 = _mm_tab_fused(acc1, den1, bp1, Wp2, 4)
    acc2, den2 = _edge_phase(tab2, as2, ad2, src2d, dst2d, ep_real, 4)
    tab3 = _mm_tab_fused(acc2, den2, bp2, Wp3, 6)
    acc3, den3 = _edge_phase(tab3, as3, ad3, src2d, dst2d, ep_real, 6)

    out = acc3[:, :n, 0:7] / (den3[:, :n] + 1e-16)[:, :, None]
    out = out.mean(axis=0) + b3
    out = jnp.where(out > 0, out, jnp.expm1(out))
    return jax.nn.log_softmax(out, axis=1)


# speedup vs baseline: 76.1106x; 1.0366x over previous
import functools

import jax
import jax.numpy as jnp
from jax import lax
from jax.experimental import pallas as pl
from jax.experimental.pallas import tpu as pltpu
from jax.experimental.pallas import tpu_sc as plsc

NSC = 2
NTILE = 16
LANES = 16
K_CHUNK = 384
IDX_B = 128
IDX_N = K_CHUNK // IDX_B
BLOCK_M = 512


def _mm_tab_body(heads, x_ref, w_ref, tab_ref):
    res = jnp.dot(x_ref[...], w_ref[...], preferred_element_type=jnp.float32)
    for h in range(heads):
        tab_ref[h, :, :] = res[:, h * LANES:(h + 1) * LANES]


def _mm_tab(x, w, n_pad, heads):
    m, k = x.shape
    return pl.pallas_call(
        functools.partial(_mm_tab_body, heads),
        grid=(n_pad // BLOCK_M,),
        in_specs=[
            pl.BlockSpec((BLOCK_M, k), lambda i: (i, 0)),
            pl.BlockSpec((k, heads * LANES), lambda i: (0, 0)),
        ],
        out_specs=pl.BlockSpec((heads, BLOCK_M, LANES), lambda i: (0, i, 0)),
        out_shape=jax.ShapeDtypeStruct((heads, n_pad, LANES), jnp.float32),
    )(x, w)


def _mm_tab_fused_body(heads_in, heads, acc_ref, den_ref, b_ref, w_ref,
                       tab_ref):
    xs = []
    for h in range(heads_in):
        xh = (acc_ref[h] / (den_ref[h][:, None] + 1e-16)
              + b_ref[h][None, :])
        xs.append(jnp.where(xh > 0, xh, jnp.exp(xh) - 1.0))
    xb = jnp.concatenate(xs, axis=1)
    res = jnp.dot(xb, w_ref[...], preferred_element_type=jnp.float32)
    for h in range(heads):
        tab_ref[h, :, :] = res[:, h * LANES:(h + 1) * LANES]


def _mm_tab_fused(acc, den, bias, w, heads):
    heads_in, n_pad, _ = acc.shape
    return pl.pallas_call(
        functools.partial(_mm_tab_fused_body, heads_in, heads),
        grid=(n_pad // BLOCK_M,),
        in_specs=[
            pl.BlockSpec((heads_in, BLOCK_M, LANES), lambda i: (0, i, 0)),
            pl.BlockSpec((heads_in, BLOCK_M), lambda i: (0, i)),
            pl.BlockSpec((heads_in, LANES), lambda i: (0, 0)),
            pl.BlockSpec((heads_in * LANES, heads * LANES), lambda i: (0, 0)),
        ],
        out_specs=pl.BlockSpec((heads, BLOCK_M, LANES), lambda i: (0, i, 0)),
        out_shape=jax.ShapeDtypeStruct((heads, n_pad, LANES), jnp.float32),
    )(acc, den, bias, w)


@functools.lru_cache(maxsize=None)
def _make_edge_kernel(n, ep_pad, ep_real, heads):
    passes = heads // NSC
    ept = ep_pad // NTILE
    g_chunks = ept // K_CHUNK
    g2_chunks = g_chunks // 2
    nrows_t = n // NTILE
    mesh = plsc.VectorSubcoreMesh(core_axis_name="c", subcore_axis_name="s")

    idx_t = pltpu.VMEM((IDX_N, IDX_B), jnp.int32)
    row_t = pltpu.VMEM((K_CHUNK, LANES), jnp.float32)
    sca_t = pltpu.VMEM((K_CHUNK,), jnp.float32)

    @functools.partial(
        pl.kernel,
        mesh=mesh,
        compiler_params=pltpu.CompilerParams(use_tc_tiling_on_sc=False),
        out_type=(jax.ShapeDtypeStruct((heads, n, LANES), jnp.float32),
                  jax.ShapeDtypeStruct((heads, n), jnp.float32)),
        scratch_types=[
            pltpu.VMEM_SHARED((n, LANES), jnp.float32),
            pltpu.VMEM_SHARED((n,), jnp.float32),
            idx_t, idx_t, idx_t, idx_t,
            row_t, row_t,
            sca_t, sca_t, sca_t, sca_t,
            row_t,
            sca_t,
            pltpu.SemaphoreType.DMA,
            pltpu.SemaphoreType.DMA,
        ],
    )
    def ek(tab, ast, adt, src2d, dst2d, zacc, zden, acc_out, den_out,
           acc_s, den_s, srcv0, dstv0, srcv1, dstv1, rows0, rows1,
           asv0, adv0, asv1, adv1, upd, exv, gsem0, gsem1):
        c = lax.axis_index("c")
        s = lax.axis_index("s")
        r0 = s * nrows_t
        rsl = pl.ds(r0, nrows_t)
        bufs = ((srcv0, dstv0, rows0, asv0, adv0, gsem0),
                (srcv1, dstv1, rows1, asv1, adv1, gsem1))
        for p in range(passes):
            head = c + NSC * p

            def gather_copies(g, b, make_only):
                srcv, dstv, rows, asv, adv, gsem = bufs[b]
                mk = pltpu.make_async_copy if make_only else pltpu.async_copy
                cps = []
                for j in range(IDX_N):
                    sl = pl.ds(j * IDX_B, IDX_B)
                    cps.append(mk(tab.at[head].at[srcv.at[j]],
                                  rows.at[sl], gsem))
                    cps.append(mk(ast.at[head].at[srcv.at[j]],
                                  asv.at[sl], gsem))
                    cps.append(mk(adt.at[head].at[dstv.at[j]],
                                  adv.at[sl], gsem))
                return cps

            def fire(g, b):
                srcv, dstv = bufs[b][0], bufs[b][1]
                row_off = s * (ept // IDX_B) + g * IDX_N
                pltpu.sync_copy(src2d.at[pl.ds(row_off, IDX_N)], srcv)
                pltpu.sync_copy(dst2d.at[pl.ds(row_off, IDX_N)], dstv)
                gather_copies(g, b, False)

            def process(g, b):
                srcv, dstv, rows, asv, adv, gsem = bufs[b]
                for cp in gather_copies(g, b, True):
                    cp.wait()

                def inner(i, carry2):
                    base = i * LANES
                    sl = pl.ds(base, LANES)
                    alpha = asv[sl] + adv[sl]
                    ex = jnp.exp(jnp.maximum(alpha, 0.2 * alpha))
                    exv[sl] = ex
                    for jj in range(LANES):
                        r = base + jj
                        upd[r, pl.ds(0, LANES)] = (
                            rows[r, pl.ds(0, LANES)] * ex[jj])
                    return carry2

                lax.fori_loop(0, K_CHUNK // LANES, inner, 0)
                cps2 = []
                for j in range(IDX_N):
                    sl = pl.ds(j * IDX_B, IDX_B)
                    cps2.append(pltpu.async_copy(
                        upd.at[sl], acc_s.at[dstv.at[j]], add=True, sem=gsem))
                    pltpu.sync_copy(exv.at[sl], den_s.at[dstv.at[j]], add=True)
                for cp in cps2:
                    cp.wait()

            pltpu.sync_copy(zacc.at[rsl], acc_s.at[rsl])
            pltpu.sync_copy(zden.at[rsl], den_s.at[rsl])
            plsc.subcore_barrier()

            fire(0, 0)
            fire(1, 1)

            def chunk2(g2, carry):
                ga = 2 * g2
                process(ga, 0)

                @pl.when(g2 < g2_chunks - 1)
                def _():
                    fire(ga + 2, 0)

                process(ga + 1, 1)

                @pl.when(g2 < g2_chunks - 1)
                def _():
                    fire(ga + 3, 1)

                return carry

            lax.fori_loop(0, g2_chunks, chunk2, 0)
            plsc.subcore_barrier()
            pltpu.sync_copy(acc_s.at[rsl], acc_out.at[head].at[rsl])
            pltpu.sync_copy(den_s.at[rsl], den_out.at[head].at[rsl])
            plsc.subcore_barrier()

    return ek


@functools.lru_cache(maxsize=None)
def _make_edge_pair_kernel(n, ep_pad, npairs):
    ept = ep_pad // NTILE
    g_chunks = ept // K_CHUNK
    g2_chunks = g_chunks // 2
    nrows_t = n // NTILE
    passes = (npairs + NSC - 1) // NSC
    mesh = plsc.VectorSubcoreMesh(core_axis_name="c", subcore_axis_name="s")

    idx_t = pltpu.VMEM((IDX_N, IDX_B), jnp.int32)
    row_t = pltpu.VMEM((K_CHUNK, LANES), jnp.float32)
    sca_t = pltpu.VMEM((K_CHUNK,), jnp.float32)

    @functools.partial(
        pl.kernel,
        mesh=mesh,
        compiler_params=pltpu.CompilerParams(use_tc_tiling_on_sc=False),
        out_type=jax.ShapeDtypeStruct((npairs, n, LANES), jnp.float32),
        scratch_types=[
            pltpu.VMEM_SHARED((n, LANES), jnp.float32),
            idx_t, idx_t, idx_t, idx_t,
            row_t, row_t,
            sca_t, sca_t, sca_t, sca_t,
            sca_t, sca_t, sca_t, sca_t,
            row_t,
            pltpu.SemaphoreType.DMA,
            pltpu.SemaphoreType.DMA,
        ],
    )
    def ekp(tabp, ast, adt, src2d, dst2d, zacc, acc_out,
            acc_s, srcv0, dstv0, srcv1, dstv1, rows0, rows1,
            asa0, ada0, asa1, ada1, asb0, adb0, asb1, adb1, upd,
            gsem0, gsem1):
        c = lax.axis_index("c")
        s = lax.axis_index("s")
        r0 = s * nrows_t
        rsl = pl.ds(r0, nrows_t)
        bufs = ((srcv0, dstv0, rows0, asa0, ada0, asb0, adb0, gsem0),
                (srcv1, dstv1, rows1, asa1, ada1, asb1, adb1, gsem1))
        iota = lax.iota(jnp.int32, LANES)
        for pp in range(passes):
            q = c + NSC * pp

            def gather_copies(b, q, make_only):
                srcv, dstv, rows, asa, ada, asb, adb, gsem = bufs[b]
                mk = pltpu.make_async_copy if make_only else pltpu.async_copy
                cps = []
                for j in range(IDX_N):
                    sl = pl.ds(j * IDX_B, IDX_B)
                    cps.append(mk(tabp.at[q].at[srcv.at[j]],
                                  rows.at[sl], gsem))
                    cps.append(mk(ast.at[2 * q].at[srcv.at[j]],
                                  asa.at[sl], gsem))
                    cps.append(mk(adt.at[2 * q].at[dstv.at[j]],
                                  ada.at[sl], gsem))
                    cps.append(mk(ast.at[2 * q + 1].at[srcv.at[j]],
                                  asb.at[sl], gsem))
                    cps.append(mk(adt.at[2 * q + 1].at[dstv.at[j]],
                                  adb.at[sl], gsem))
                return cps

            def fire(g, b, q):
                srcv, dstv = bufs[b][0], bufs[b][1]
                row_off = s * (ept // IDX_B) + g * IDX_N
                pltpu.sync_copy(src2d.at[pl.ds(row_off, IDX_N)], srcv)
                pltpu.sync_copy(dst2d.at[pl.ds(row_off, IDX_N)], dstv)
                gather_copies(b, q, False)

            def process(g, b, q):
                srcv, dstv, rows, asa, ada, asb, adb, gsem = bufs[b]
                for cp in gather_copies(b, q, True):
                    cp.wait()

                def inner(i, carry2):
                    base = i * LANES
                    sl = pl.ds(base, LANES)
                    aa = asa[sl] + ada[sl]
                    exa = jnp.exp(jnp.maximum(aa, 0.2 * aa))
                    ab = asb[sl] + adb[sl]
                    exb = jnp.exp(jnp.maximum(ab, 0.2 * ab))
                    for jj in range(LANES):
                        r = base + jj
                        exv16 = jnp.where(iota < 8, exa[jj], exb[jj])
                        upd[r, pl.ds(0, LANES)] = (
                            rows[r, pl.ds(0, LANES)] * exv16)
                    return carry2

                lax.fori_loop(0, K_CHUNK // LANES, inner, 0)
                cps2 = []
                for j in range(IDX_N):
                    sl = pl.ds(j * IDX_B, IDX_B)
                    cps2.append(pltpu.async_copy(
                        upd.at[sl], acc_s.at[dstv.at[j]], add=True,
                        sem=gsem))
                for cp in cps2:
                    cp.wait()

            @pl.when(q < npairs)
            def _():
                pltpu.sync_copy(zacc.at[rsl], acc_s.at[rsl])
                plsc.subcore_barrier()
                fire(0, 0, q)
                fire(1, 1, q)

                def chunk2(g2, carry):
                    ga = 2 * g2
                    process(ga, 0, q)

                    @pl.when(g2 < g2_chunks - 1)
                    def _():
                        fire(ga + 2, 0, q)

                    process(ga + 1, 1, q)

                    @pl.when(g2 < g2_chunks - 1)
                    def _():
                        fire(ga + 3, 1, q)

                    return carry

                lax.fori_loop(0, g2_chunks, chunk2, 0)
                plsc.subcore_barrier()
                pltpu.sync_copy(acc_s.at[rsl], acc_out.at[q].at[rsl])
                plsc.subcore_barrier()

    return ekp


def _pad_weights(W, att_src, att_dst, bias, heads, out_ch):
    cpad = LANES - out_ch
    Wp = jnp.pad(W.reshape(W.shape[0], heads, out_ch), ((0, 0), (0, 0), (0, cpad)))
    ap_s = jnp.pad(att_src, ((0, 0), (0, cpad)))
    ap_d = jnp.pad(att_dst, ((0, 0), (0, cpad)))
    if bias.shape[0] == heads * out_ch:
        bp = jnp.pad(bias.reshape(heads, out_ch), ((0, 0), (0, cpad)))
    else:
        bp = None
    return Wp.reshape(W.shape[0], heads * LANES), ap_s, ap_d, bp


def _edge_phase(tab, ap_s, ap_d, src2d, dst2d, ep_real, heads):
    n_pad = tab.shape[1]
    ep_pad = src2d.shape[0] * IDX_B
    ast = (tab * ap_s[:, None, :]).sum(-1)
    adt = (tab * ap_d[:, None, :]).sum(-1)
    zacc = jnp.zeros((n_pad, LANES), jnp.float32)
    zden = jnp.zeros((n_pad,), jnp.float32)
    ek = _make_edge_kernel(n_pad, ep_pad, ep_real, heads)
    return ek(tab, ast, adt, src2d, dst2d, zacc, zden)


def kernel(x, edge_index, W1, asrc1, adst1, b1, W2, asrc2, adst2, b2, W3,
           asrc3, adst3, b3):
    n = x.shape[0]
    n_pad = ((n + BLOCK_M - 1) // BLOCK_M) * BLOCK_M
    e = edge_index.shape[1]
    ep_real = e + n
    epg = NTILE * K_CHUNK * 2
    ep_pad = ((ep_real + epg - 1) // epg) * epg
    loop = jnp.arange(n, dtype=edge_index.dtype)
    padv = n + (jnp.arange(ep_pad - ep_real, dtype=edge_index.dtype)
                % (n_pad - n))
    src2d = jnp.concatenate([edge_index[0], loop, padv]).reshape(-1, IDX_B)
    dst2d = jnp.concatenate([edge_index[1], loop, padv]).reshape(-1, IDX_B)

    Wp1, as1, ad1, bp1 = _pad_weights(W1, asrc1, adst1, b1, 4, 16)
    Wp2, as2, ad2, bp2 = _pad_weights(W2, asrc2, adst2, b2, 4, 16)
    Wp3, as3, ad3, _ = _pad_weights(W3, asrc3, adst3, b3, 6, 7)

    tab1 = _mm_tab(x, Wp1, n_pad, 4)
    acc1, den1 = _edge_phase(tab1, as1, ad1, src2d, dst2d, ep_real, 4)
    tab2 = _mm_tab_fused(acc1, den1, bp1, Wp2, 4)
    acc2, den2 = _edge_phase(tab2, as2, ad2, src2d, dst2d, ep_real, 4)
    tab3 = _mm_tab_fused(acc2, den2, bp2, Wp3, 6)
    ast3 = (tab3 * as3[:, None, :]).sum(-1)
    adt3 = (tab3 * ad3[:, None, :]).sum(-1)
    ones = jnp.ones((3, n_pad, 1), jnp.float32)
    tabp = jnp.concatenate(
        [tab3[0::2, :, 0:7], ones, tab3[1::2, :, 0:7], ones], axis=2)
    zacc = jnp.zeros((n_pad, LANES), jnp.float32)
    ekp = _make_edge_pair_kernel(n_pad, ep_pad, 3)
    accp = ekp(tabp, ast3, adt3, src2d, dst2d, zacc)
    num = jnp.stack([accp[:, :n, 0:7], accp[:, :n, 8:15]])
    den = jnp.stack([accp[:, :n, 7], accp[:, :n, 15]])
    out = (num / (den + 1e-16)[:, :, :, None]).mean(axis=(0, 1)) + b3
    out = jnp.where(out > 0, out, jnp.expm1(out))
    return jax.nn.log_softmax(out, axis=1)
